# baseline (speedup 1.0000x reference)
import functools
import math

import jax
import jax.numpy as jnp
from jax import lax
from jax.experimental import pallas as pl
from jax.experimental.pallas import tpu as pltpu
from jax.experimental.pallas import tpu_sc as plsc

D_MODEL = 768
_SCALE = math.sqrt(D_MODEL)

_info = plsc.get_sparse_core_info()
_NC = _info.num_cores
_NS = _info.num_subcores
_L = _info.num_lanes
_NW = _NC * _NS

_CHUNK = 32
_NG = 2
_NO = 3
_PERIOD = 6


def _make_kernel(B: int):
    assert B % (_NW * _CHUNK) == 0
    b_per_w = B // _NW
    n_chunks = b_per_w // _CHUNK
    n_vecs = D_MODEL // _L
    loop_iters = (n_chunks - _NO - _NO) // _PERIOD
    loop_end = _NO + _PERIOD * loop_iters
    assert loop_end + _NG <= n_chunks

    mesh = plsc.VectorSubcoreMesh(core_axis_name="c", subcore_axis_name="s")

    @functools.partial(
        pl.kernel,
        mesh=mesh,
        out_type=jax.ShapeDtypeStruct((B, D_MODEL), jnp.float32),
        scratch_types=(
            [pltpu.VMEM((n_chunks, _CHUNK), jnp.int32)]
            + [pltpu.VMEM((_CHUNK, D_MODEL), jnp.float32)] * (_NG + _NO)
            + [pltpu.SemaphoreType.DMA] * (_NG + _NO)
        ),
    )
    def emb_kernel(table_hbm, x_hbm, out_hbm, idx_v, *rest):
        gbufs = rest[:_NG]
        obufs = rest[_NG:_NG + _NO]
        gsems = rest[_NG + _NO:2 * _NG + _NO]
        osems = rest[2 * _NG + _NO:2 * (_NG + _NO)]

        wid = lax.axis_index("s") * _NC + lax.axis_index("c")
        base = wid * b_per_w

        pltpu.sync_copy(x_hbm.at[wid], idx_v)

        def issue_gather(c, g):
            pltpu.async_copy(table_hbm.at[idx_v.at[c]], gbufs[g], gsems[g])

        def wait_gather(g):
            pltpu.make_async_copy(
                table_hbm.at[idx_v.at[0]], gbufs[g], gsems[g]).wait()

        def issue_write(c, o):
            pltpu.async_copy(
                obufs[o], out_hbm.at[pl.ds(base + c * _CHUNK, _CHUNK)],
                osems[o])

        def wait_write(o):
            pltpu.make_async_copy(
                obufs[o], out_hbm.at[pl.ds(base, _CHUNK)], osems[o]).wait()

        def scale(g, o):
            src = gbufs[g]
            dst = obufs[o]
            def row_body(r, carry):
                for j in range(n_vecs):
                    sl = (r, pl.ds(j * _L, _L))
                    dst[sl] = src[sl] * _SCALE
                return carry
            lax.fori_loop(0, _CHUNK, row_body, 0)

        def process(c, g, o, wait_w, issue_next):
            wait_gather(g)
            if wait_w:
                wait_write(o)
            scale(g, o)
            if issue_next:
                issue_gather(c + _NG, g)
            issue_write(c, o)

        for g in range(_NG):
            issue_gather(g, g)
        for c in range(_NO):
            process(c, c % _NG, c % _NO, False, True)

        def loop_body(i, carry):
            cc = _NO + _PERIOD * i
            for j in range(_PERIOD):
                c = cc + j
                process(c, (_NO + j) % _NG, j % _NO, True, True)
            return carry
        lax.fori_loop(0, loop_iters, loop_body, 0)

        for c in range(loop_end, n_chunks):
            process(c, c % _NG, c % _NO, True, c + _NG < n_chunks)
        for o in range(_NO):
            wait_write(o)

    return emb_kernel


def kernel(table, x):
    B = x.size
    x_blocked = x.reshape(_NW, B // _NW // _CHUNK, _CHUNK)
    out = _make_kernel(B)(table, x_blocked)
    return out.reshape(x.shape + (D_MODEL,))
